# Initial kernel scaffold; baseline (speedup 1.0000x reference)
import jax
import jax.numpy as jnp
from jax.experimental import pallas as pl


def kernel(x, batch):
    raise NotImplementedError("write your pallas kernel here")


# speedup vs baseline: 4.5032x; 4.5032x over previous
import functools

import jax
import jax.numpy as jnp
from jax import lax
from jax.experimental import pallas as pl
from jax.experimental.pallas import tpu as pltpu
from jax.experimental.pallas import tpu_sc as plsc

N = 320000
D = 128
S = 512

NC = 2
NS = 16
DC = D // NC
ROWS_PER_SUB = N // NS
CHUNK = 1000
SUB = 100
N_CHUNKS = ROWS_PER_SUB // CHUNK
N_SUB = CHUNK // SUB
ROWS_PER_OUT = S // NS


@functools.partial(
    pl.kernel,
    out_type=jax.ShapeDtypeStruct((S, D), jnp.float32),
    mesh=plsc.VectorSubcoreMesh(core_axis_name="c", subcore_axis_name="s"),
    scratch_types=[
        pltpu.VMEM((CHUNK, DC), jnp.float32),
        pltpu.VMEM((N_SUB, SUB), jnp.int32),
        pltpu.VMEM((ROWS_PER_OUT, DC), jnp.float32),
        pltpu.VMEM_SHARED((S, DC), jnp.float32),
    ],
    compiler_params=pltpu.CompilerParams(use_tc_tiling_on_sc=False),
)
def _seg_sum(x_hbm, ids_hbm, zero_hbm, out_hbm, buf, idsv, obuf, acc):
    c = lax.axis_index("c")
    s = lax.axis_index("s")
    col0 = c * DC
    row0 = s * ROWS_PER_SUB

    @pl.when(s == 0)
    def _():
        pltpu.sync_copy(zero_hbm.at[:, pl.ds(col0, DC)], acc)

    plsc.subcore_barrier()

    def chunk_body(j, carry):
        pltpu.sync_copy(
            x_hbm.at[pl.ds(row0 + j * CHUNK, CHUNK), pl.ds(col0, DC)], buf
        )
        pltpu.sync_copy(ids_hbm.at[pl.ds(s * (ROWS_PER_SUB // SUB) + j * N_SUB, N_SUB)], idsv)
        for k in range(N_SUB):
            pltpu.sync_copy(
                buf.at[pl.ds(k * SUB, SUB)], acc.at[idsv.at[k]], add=True
            )
        return carry

    lax.fori_loop(0, N_CHUNKS, chunk_body, 0)

    plsc.subcore_barrier()

    pltpu.sync_copy(acc.at[pl.ds(s * ROWS_PER_OUT, ROWS_PER_OUT)], obuf)
    pltpu.sync_copy(
        obuf, out_hbm.at[pl.ds(s * ROWS_PER_OUT, ROWS_PER_OUT), pl.ds(col0, DC)]
    )


def kernel(x, batch):
    ids = batch.astype(jnp.int32).reshape(N // SUB, SUB)
    zero = jnp.zeros((S, D), jnp.float32)
    return _seg_sum(x, ids, zero)


# speedup vs baseline: 5.0113x; 1.1128x over previous
import functools

import jax
import jax.numpy as jnp
from jax import lax
from jax.experimental import pallas as pl
from jax.experimental.pallas import tpu as pltpu
from jax.experimental.pallas import tpu_sc as plsc

N = 320000
D = 128
S = 512

NC = 2
NS = 16
DC = D // NC
ROWS_PER_SUB = N // NS
CHUNK = 500
SUB = 100
N_CHUNKS = ROWS_PER_SUB // CHUNK
N_SUB = CHUNK // SUB
IDS_PER_SUB = ROWS_PER_SUB // SUB
ROWS_PER_OUT = S // NS


@functools.partial(
    pl.kernel,
    out_type=jax.ShapeDtypeStruct((NC, S, DC), jnp.float32),
    mesh=plsc.VectorSubcoreMesh(core_axis_name="c", subcore_axis_name="s"),
    scratch_types=[
        pltpu.VMEM((CHUNK, DC), jnp.float32),
        pltpu.VMEM((CHUNK, DC), jnp.float32),
        pltpu.VMEM((IDS_PER_SUB, SUB), jnp.int32),
        pltpu.VMEM((ROWS_PER_OUT, DC), jnp.float32),
        pltpu.VMEM_SHARED((S, DC), jnp.float32),
        pltpu.SemaphoreType.DMA,
        pltpu.SemaphoreType.DMA,
        pltpu.SemaphoreType.DMA,
        pltpu.SemaphoreType.DMA,
    ],
    compiler_params=pltpu.CompilerParams(use_tc_tiling_on_sc=False),
)
def _seg_sum(x_hbm, ids_hbm, out_hbm, buf_a, buf_b, idsv, obuf,
             acc, lsem_a, lsem_b, ssem_a, ssem_b):
    c = lax.axis_index("c")
    s = lax.axis_index("s")
    col0 = c * DC
    row0 = s * ROWS_PER_SUB

    bufs = (buf_a, buf_b)
    lsems = (lsem_a, lsem_b)
    ssems = (ssem_a, ssem_b)

    zvec = jnp.zeros((16,), jnp.float32)
    for r in range(ROWS_PER_OUT):
        for k in range(DC // 16):
            obuf[r, pl.ds(k * 16, 16)] = zvec
    pltpu.sync_copy(obuf, acc.at[pl.ds(s * ROWS_PER_OUT, ROWS_PER_OUT)])

    pltpu.sync_copy(ids_hbm.at[pl.ds(s * IDS_PER_SUB, IDS_PER_SUB)], idsv)

    plsc.subcore_barrier()

    def load(j, p):
        return pltpu.async_copy(
            x_hbm.at[pl.ds(row0 + j * CHUNK, CHUNK), pl.ds(col0, DC)],
            bufs[p], lsems[p],
        )

    loadd = [None, None]
    scatd = [None, None]
    loadd[0] = load(0, 0)
    for j in range(N_CHUNKS):
        p = j % 2
        q = 1 - p
        if j >= 1:
            for d in scatd[q]:
                d.wait()
        if j + 1 < N_CHUNKS:
            loadd[q] = load(j + 1, q)
        loadd[p].wait()
        scatd[p] = [
            pltpu.async_copy(
                bufs[p].at[pl.ds(k * SUB, SUB)],
                acc.at[idsv.at[j * N_SUB + k]],
                ssems[p], add=True,
            )
            for k in range(N_SUB)
        ]
    for d in scatd[(N_CHUNKS - 1) % 2]:
        d.wait()

    plsc.subcore_barrier()

    pltpu.sync_copy(acc.at[pl.ds(s * ROWS_PER_OUT, ROWS_PER_OUT)], obuf)
    pltpu.sync_copy(
        obuf, out_hbm.at[c, pl.ds(s * ROWS_PER_OUT, ROWS_PER_OUT)]
    )


def kernel(x, batch):
    ids = batch.astype(jnp.int32).reshape(N // SUB, SUB)
    halves = _seg_sum(x, ids)
    return jnp.concatenate([halves[0], halves[1]], axis=1)


# speedup vs baseline: 7.2133x; 1.4394x over previous
import functools

import jax
import jax.numpy as jnp
from jax import lax
from jax.experimental import pallas as pl
from jax.experimental.pallas import tpu as pltpu
from jax.experimental.pallas import tpu_sc as plsc

N = 320000
D = 128
S = 512

NC = 2
NS = 16
DC = D // NC
ROWS_PER_SUB = N // NS
CHUNK = 400
GROUP = 16
N_CHUNKS = ROWS_PER_SUB // CHUNK
N_GROUPS = CHUNK // GROUP
ROWS_PER_OUT = S // NS
SCAT = 128
N_SCAT = S // SCAT
NV = DC // 16


@functools.partial(
    pl.kernel,
    out_type=jax.ShapeDtypeStruct((NC, S, DC), jnp.float32),
    mesh=plsc.VectorSubcoreMesh(core_axis_name="c", subcore_axis_name="s"),
    scratch_types=[
        pltpu.VMEM((CHUNK, DC), jnp.float32),
        pltpu.VMEM((CHUNK, DC), jnp.float32),
        pltpu.VMEM((1, CHUNK), jnp.int32),
        pltpu.VMEM((1, CHUNK), jnp.int32),
        pltpu.VMEM((ROWS_PER_OUT, DC), jnp.float32),
        pltpu.VMEM((S, DC), jnp.float32),
        pltpu.VMEM((N_SCAT, SCAT), jnp.int32),
        pltpu.VMEM_SHARED((S, DC), jnp.float32),
        pltpu.SemaphoreType.DMA,
        pltpu.SemaphoreType.DMA,
        pltpu.SemaphoreType.DMA,
    ],
    compiler_params=pltpu.CompilerParams(use_tc_tiling_on_sc=False),
)
def _seg_sum(x_hbm, ids_hbm, out_hbm, buf_a, buf_b, idb_a, idb_b, obuf,
             lacc, sidx, acc, lsem_a, lsem_b, ssem):
    c = lax.axis_index("c")
    s = lax.axis_index("s")
    col0 = c * DC
    row0 = s * ROWS_PER_SUB
    idrow0 = s * N_CHUNKS

    bufs = (buf_a, buf_b)
    idbs = (idb_a, idb_b)
    lsems = (lsem_a, lsem_b)
    zvec = jnp.zeros((16,), jnp.float32)

    for r in range(ROWS_PER_OUT):
        for k in range(NV):
            obuf[r, pl.ds(k * 16, 16)] = zvec
    pltpu.sync_copy(obuf, acc.at[pl.ds(s * ROWS_PER_OUT, ROWS_PER_OUT)])

    def zero_body(r, carry):
        for k in range(NV):
            lacc[r, pl.ds(k * 16, 16)] = zvec
        return carry

    lax.fori_loop(0, S, zero_body, 0)

    for r in range(N_SCAT):
        for k in range(SCAT // 16):
            sidx[r, pl.ds(k * 16, 16)] = (
                lax.iota(jnp.int32, 16) + (r * SCAT + k * 16)
            )

    def load(j, p):
        pltpu.async_copy(
            x_hbm.at[pl.ds(row0 + j * CHUNK, CHUNK), pl.ds(col0, DC)],
            bufs[p], lsems[p],
        )
        pltpu.async_copy(
            ids_hbm.at[pl.ds(idrow0 + j, 1)], idbs[p], lsems[p],
        )

    def wait_load(p):
        pltpu.make_async_copy(
            x_hbm.at[pl.ds(row0, CHUNK), pl.ds(col0, DC)], bufs[p], lsems[p]
        ).wait()
        pltpu.make_async_copy(
            ids_hbm.at[pl.ds(idrow0, 1)], idbs[p], lsems[p]
        ).wait()

    def process_chunk(buf, idb):
        def group_body(g, carry):
            base = g * GROUP
            gv = idb[0, pl.ds(base, GROUP)]
            id_first = gv[0]
            id_last = gv[GROUP - 1]

            def fast():
                for k in range(NV):
                    cs = pl.ds(k * 16, 16)
                    t0 = [buf[base + i, cs] + buf[base + i + 8, cs]
                          for i in range(8)]
                    t1 = [t0[i] + t0[i + 4] for i in range(4)]
                    t2 = [t1[0] + t1[2], t1[1] + t1[3]]
                    plsc.addupdate(lacc.at[id_first, cs], t2[0] + t2[1])

            def slow():
                for i in range(GROUP):
                    rid = gv[i]
                    for k in range(NV):
                        cs = pl.ds(k * 16, 16)
                        plsc.addupdate(lacc.at[rid, cs], buf[base + i, cs])

            lax.cond(id_first == id_last, fast, slow)
            return carry

        lax.fori_loop(0, N_GROUPS, group_body, 0)

    load(0, 0)

    def pair_body(g, carry):
        j0 = 2 * g
        load(j0 + 1, 1)
        wait_load(0)
        process_chunk(bufs[0], idbs[0])
        load(jnp.minimum(j0 + 2, N_CHUNKS - 1), 0)
        wait_load(1)
        process_chunk(bufs[1], idbs[1])
        return carry

    lax.fori_loop(0, N_CHUNKS // 2, pair_body, 0)
    wait_load(0)

    scatd = [
        pltpu.async_copy(
            lacc.at[pl.ds(r * SCAT, SCAT)], acc.at[sidx.at[r]], ssem,
            add=True,
        )
        for r in range(N_SCAT)
    ]
    for d in scatd:
        d.wait()

    plsc.subcore_barrier()

    pltpu.sync_copy(acc.at[pl.ds(s * ROWS_PER_OUT, ROWS_PER_OUT)], obuf)
    pltpu.sync_copy(
        obuf, out_hbm.at[c, pl.ds(s * ROWS_PER_OUT, ROWS_PER_OUT)]
    )


def kernel(x, batch):
    ids = batch.astype(jnp.int32).reshape(N // CHUNK, CHUNK)
    halves = _seg_sum(x, ids)
    return jnp.concatenate([halves[0], halves[1]], axis=1)
